# baseline (speedup 1.0000x reference)
import functools

import jax
import jax.numpy as jnp
from jax import lax
from jax.experimental import pallas as pl
from jax.experimental.pallas import tpu as pltpu
from jax.experimental.pallas import tpu_sc as plsc

_VOCAB = 1000000
_EMB = 64
_CTX = 16
_RF = 8
_B = 16384
_T = _CTX + _RF - 1
_IC = 128


def _make_sc_gather(B):
    info = plsc.get_sparse_core_info()
    NC, NS = info.num_cores, info.num_subcores
    NW = NC * NS
    b_per_w = B // NW
    mesh = plsc.VectorSubcoreMesh(core_axis_name="c", subcore_axis_name="s")

    @functools.partial(
        pl.kernel,
        mesh=mesh,
        out_type=jax.ShapeDtypeStruct((B, _EMB), jnp.float32),
        scratch_types=[
            pltpu.VMEM((b_per_w,), jnp.int32),
            pltpu.VMEM((b_per_w, _EMB), jnp.float32),
            pltpu.SemaphoreType.DMA,
        ],
        compiler_params=pltpu.CompilerParams(use_tc_tiling_on_sc=False),
    )
    def gather_k(table_hbm, idx_hbm, out_hbm, idx_v, rows_v, sem):
        wid = lax.axis_index("s") * NC + lax.axis_index("c")
        base = wid * b_per_w
        pltpu.sync_copy(idx_hbm.at[pl.ds(base, b_per_w)], idx_v)
        copies = []
        for q in range(b_per_w // _IC):
            sl = pl.ds(q * _IC, _IC)
            copies.append(
                pltpu.async_copy(table_hbm.at[idx_v.at[sl]], rows_v.at[sl], sem)
            )
        for cp in copies:
            cp.wait()
        pltpu.sync_copy(rows_v, out_hbm.at[pl.ds(base, b_per_w)])

    return gather_k


def _tc_main_body(st_ref, wct_ref, out_ref, m0_ref):
    st = st_ref[...]
    inv_rf = 1.0 / _RF
    w = st[0]
    for t in range(1, _RF):
        w = w + st[t]
    m0_ref[...] = w * inv_rf
    for j in range(1, _CTX):
        w = w - st[j - 1] + st[j - 1 + _RF]
        m = w * inv_rf
        c = jnp.concatenate([st[j - 1], m], axis=0)
        y = jnp.tanh(
            jnp.dot(wct_ref[j], c, preferred_element_type=jnp.float32)
        )
        out_ref[j] = y
    out_ref[_CTX:] = st[_CTX:]


def _tc_main_call(st_t, WcT, bn):
    n_blocks = _B // bn
    return pl.pallas_call(
        _tc_main_body,
        grid=(n_blocks,),
        in_specs=[
            pl.BlockSpec((_T, _EMB, bn), lambda i: (0, 0, i)),
            pl.BlockSpec((_CTX, _EMB, 2 * _EMB), lambda i: (0, 0, 0)),
        ],
        out_specs=[
            pl.BlockSpec((_T, _EMB, bn), lambda i: (0, 0, i)),
            pl.BlockSpec((_EMB, bn), lambda i: (0, i)),
        ],
        out_shape=[
            jax.ShapeDtypeStruct((_T, _EMB, _B), jnp.float32),
            jax.ShapeDtypeStruct((_EMB, _B), jnp.float32),
        ],
        compiler_params=pltpu.CompilerParams(
            dimension_semantics=("arbitrary",),
        ),
    )(st_t, WcT)


def _tc_block0_body(emb_ref, m0_ref, w1t_ref, w2t_ref, out_ref):
    out_ref[...] = jnp.tanh(
        lax.dot_general(
            w1t_ref[...],
            emb_ref[...],
            (((1,), (1,)), ((), ())),
            preferred_element_type=jnp.float32,
        )
        + jnp.dot(w2t_ref[...], m0_ref[...], preferred_element_type=jnp.float32)
    )


def _tc_block0_call(emb, m0, W1_0T, W2_0T, bn):
    n_blocks = _B // bn
    return pl.pallas_call(
        _tc_block0_body,
        grid=(n_blocks,),
        in_specs=[
            pl.BlockSpec((bn, _EMB), lambda i: (i, 0)),
            pl.BlockSpec((_EMB, bn), lambda i: (0, i)),
            pl.BlockSpec((_EMB, _EMB), lambda i: (0, 0)),
            pl.BlockSpec((_EMB, _EMB), lambda i: (0, 0)),
        ],
        out_specs=pl.BlockSpec((_EMB, bn), lambda i: (0, i)),
        out_shape=jax.ShapeDtypeStruct((_EMB, _B), jnp.float32),
        compiler_params=pltpu.CompilerParams(
            dimension_semantics=("arbitrary",),
        ),
    )(emb, m0, W1_0T, W2_0T)


def kernel(emb_table, network_state, W1, W2, input_indices):
    idx = input_indices[:, 0]
    emb = _make_sc_gather(_B)(emb_table, idx)
    WcT = jnp.transpose(jnp.concatenate([W1, W2], axis=1), (0, 2, 1))
    st_t = jnp.transpose(network_state, (2, 1, 0))
    out_t, m0 = _tc_main_call(st_t, WcT, bn=512)
    y0 = _tc_block0_call(
        emb, m0, jnp.transpose(W1[0]), jnp.transpose(W2[0]), bn=512
    )
    out_t = out_t.at[0].set(y0)
    return jnp.transpose(out_t, (2, 1, 0))


# speedup vs baseline: 1.0248x; 1.0248x over previous
import functools

import jax
import jax.numpy as jnp
from jax import lax
from jax.experimental import pallas as pl
from jax.experimental.pallas import tpu as pltpu
from jax.experimental.pallas import tpu_sc as plsc

_VOCAB = 1000000
_EMB = 64
_CTX = 16
_RF = 8
_B = 16384
_T = _CTX + _RF - 1
_IC = 128


def _make_sc_gather(B):
    info = plsc.get_sparse_core_info()
    NC, NS = info.num_cores, info.num_subcores
    NW = NC * NS
    b_per_w = B // NW
    mesh = plsc.VectorSubcoreMesh(core_axis_name="c", subcore_axis_name="s")

    @functools.partial(
        pl.kernel,
        mesh=mesh,
        out_type=jax.ShapeDtypeStruct((B, _EMB), jnp.float32),
        scratch_types=[
            pltpu.VMEM((b_per_w,), jnp.int32),
            pltpu.VMEM((b_per_w, _EMB), jnp.float32),
            pltpu.SemaphoreType.DMA,
        ],
        compiler_params=pltpu.CompilerParams(use_tc_tiling_on_sc=False),
    )
    def gather_k(table_hbm, idx_hbm, out_hbm, idx_v, rows_v, sem):
        wid = lax.axis_index("s") * NC + lax.axis_index("c")
        base = wid * b_per_w
        pltpu.sync_copy(idx_hbm.at[pl.ds(base, b_per_w)], idx_v)
        copies = []
        for q in range(b_per_w // _IC):
            sl = pl.ds(q * _IC, _IC)
            copies.append(
                pltpu.async_copy(table_hbm.at[idx_v.at[sl]], rows_v.at[sl], sem)
            )
        for cp in copies:
            cp.wait()
        pltpu.sync_copy(rows_v, out_hbm.at[pl.ds(base, b_per_w)])

    return gather_k


def _tc_main_body(st_ref, wct_ref, out_ref, m0_ref):
    st = st_ref[...]
    inv_rf = 1.0 / _RF
    w = st[0]
    for t in range(1, _RF):
        w = w + st[t]
    m0_ref[...] = w * inv_rf
    for j in range(1, _CTX):
        w = w - st[j - 1] + st[j - 1 + _RF]
        m = w * inv_rf
        c = jnp.concatenate([st[j - 1], m], axis=0)
        y = jnp.tanh(
            jnp.dot(wct_ref[j], c, preferred_element_type=jnp.float32)
        )
        out_ref[j] = y
    out_ref[_CTX:] = st[_CTX:]


def _tc_main_call(st_t, WcT, bn):
    n_blocks = _B // bn
    return pl.pallas_call(
        _tc_main_body,
        grid=(n_blocks,),
        in_specs=[
            pl.BlockSpec((_T, _EMB, bn), lambda i: (0, 0, i)),
            pl.BlockSpec((_CTX, _EMB, 2 * _EMB), lambda i: (0, 0, 0)),
        ],
        out_specs=[
            pl.BlockSpec((_T, _EMB, bn), lambda i: (0, 0, i)),
            pl.BlockSpec((_EMB, bn), lambda i: (0, i)),
        ],
        out_shape=[
            jax.ShapeDtypeStruct((_T, _EMB, _B), jnp.float32),
            jax.ShapeDtypeStruct((_EMB, _B), jnp.float32),
        ],
        compiler_params=pltpu.CompilerParams(
            dimension_semantics=("arbitrary",),
        ),
    )(st_t, WcT)


def _tc_block0_body(emb_ref, m0_ref, w1t_ref, w2t_ref, out_ref):
    out_ref[...] = jnp.tanh(
        lax.dot_general(
            w1t_ref[...],
            emb_ref[...],
            (((1,), (1,)), ((), ())),
            preferred_element_type=jnp.float32,
        )
        + jnp.dot(w2t_ref[...], m0_ref[...], preferred_element_type=jnp.float32)
    )


def _tc_block0_call(emb, m0, W1_0T, W2_0T, bn):
    n_blocks = _B // bn
    return pl.pallas_call(
        _tc_block0_body,
        grid=(n_blocks,),
        in_specs=[
            pl.BlockSpec((bn, _EMB), lambda i: (i, 0)),
            pl.BlockSpec((_EMB, bn), lambda i: (0, i)),
            pl.BlockSpec((_EMB, _EMB), lambda i: (0, 0)),
            pl.BlockSpec((_EMB, _EMB), lambda i: (0, 0)),
        ],
        out_specs=pl.BlockSpec((_EMB, bn), lambda i: (0, i)),
        out_shape=jax.ShapeDtypeStruct((_EMB, _B), jnp.float32),
        compiler_params=pltpu.CompilerParams(
            dimension_semantics=("arbitrary",),
        ),
    )(emb, m0, W1_0T, W2_0T)


def kernel(emb_table, network_state, W1, W2, input_indices):
    idx = input_indices[:, 0]
    WcT = jnp.transpose(jnp.concatenate([W1, W2], axis=1), (0, 2, 1))
    st_t = jnp.transpose(network_state, (2, 1, 0))
    out_t, m0 = _tc_main_call(st_t, WcT, bn=512)
    idx, _ = lax.optimization_barrier((idx, m0))
    emb = _make_sc_gather(_B)(emb_table, idx)
    y0 = _tc_block0_call(
        emb, m0, jnp.transpose(W1[0]), jnp.transpose(W2[0]), bn=512
    )
    out_t = out_t.at[0].set(y0)
    return jnp.transpose(out_t, (2, 1, 0))
